# baseline (speedup 1.0000x reference)
import functools

import jax
import jax.numpy as jnp
from jax import lax
from jax.experimental import pallas as pl
from jax.experimental.pallas import tpu as pltpu
from jax.experimental.pallas import tpu_sc as plsc

N = 100000
D = 128
TOPK = 16
L = 16
NC = 2
NS = 16
PW = 6272
PWL = N - (NS - 1) * PW
CPW = PW // L
BIG = float(jnp.finfo(jnp.float32).max)
IMAX = 2**31 - 1

_mesh = plsc.VectorSubcoreMesh(core_axis_name="c", subcore_axis_name="s",
                               num_cores=1)


def _merge16(av, ai, xv, xi, descending):
  rv = lax.rev(xv, (0,))
  ri = lax.rev(xi, (0,))
  if descending:
    take = (av > rv) | ((av == rv) & (ai < ri))
  else:
    take = (av < rv) | ((av == rv) & (ai < ri))
  mv = jnp.where(take, av, rv)
  mi = jnp.where(take, ai, ri)
  sv, si = plsc.sort_key_val(mv, mi, descending=descending)
  return sv, si


@functools.partial(
    pl.kernel,
    out_type=(
        jax.ShapeDtypeStruct((2 * TOPK,), jnp.float32),
        jax.ShapeDtypeStruct((2 * TOPK, D), jnp.float32),
    ),
    mesh=_mesh,
    compiler_params=pltpu.CompilerParams(needs_layout_passes=False),
    scratch_types=[
        pltpu.VMEM((PW,), jnp.float32),
        pltpu.VMEM((PW + L,), jnp.float32),
        pltpu.VMEM((PW + L,), jnp.int32),
        pltpu.VMEM((PW + L,), jnp.float32),
        pltpu.VMEM((PW + L,), jnp.int32),
        pltpu.VMEM((4 * L,), jnp.float32),
        pltpu.VMEM((NS * 4 * L,), jnp.float32),
        pltpu.VMEM_SHARED((NS * 4 * L,), jnp.float32),
        pltpu.VMEM((2 * TOPK,), jnp.int32),
        pltpu.VMEM((2 * TOPK,), jnp.float32),
        pltpu.VMEM((2 * TOPK, D), jnp.float32),
        pltpu.SemaphoreType.DMA,
    ],
)
def _minmax_select(att_hbm, feat_hbm, att_out_hbm, feat_out_hbm,
                   att_v, cv_t, ci_t, cv_b, ci_b, pack_v, all_v, stage_sh,
                   idx_v, aout_v, rows_v, sem):
  cid = lax.axis_index("c")
  sid = lax.axis_index("s")

  @pl.when(cid == 0)
  def _():
    base = sid * PW
    ncnk = jnp.where(sid == NS - 1, PWL // L, CPW)
    pltpu.sync_copy(att_hbm.at[pl.ds(base, PWL)], att_v.at[pl.ds(0, PWL)])

    @pl.when(sid != NS - 1)
    def _copy_tail():
      pltpu.sync_copy(att_hbm.at[pl.ds(base + PWL, PW - PWL)],
                      att_v.at[pl.ds(PWL, PW - PWL)])

    iota = lax.iota(jnp.int32, L)
    fifteen = jnp.full((L,), 15, jnp.int32)

    def pass_a(c, carry):
      vmax, vmin = carry
      b0 = c * (2 * L)
      for u in range(2):
        v = att_v[pl.ds(b0 + u * L, L)]
        vmax = jnp.maximum(vmax, v)
        vmin = jnp.minimum(vmin, v)
      return vmax, vmin

    vmax, vmin = lax.fori_loop(
        0, ncnk // 2, pass_a,
        (jnp.full((L,), -BIG, jnp.float32), jnp.full((L,), BIG, jnp.float32)))
    thr_t = jnp.min(vmax)
    thr_b = jnp.max(vmin)

    def pass_b(c, off):
      b0 = c * (2 * L)
      for u in range(2):
        v = att_v[pl.ds(b0 + u * L, L)]
        idx = iota + (base + b0 + u * L)
        m = (v >= thr_t) | (v <= thr_b)
        pos = plsc.cumsum(m.astype(jnp.int32))
        iw = off + pos - 1
        plsc.store_scatter(cv_t, [iw], v, mask=m)
        plsc.store_scatter(ci_t, [iw], idx, mask=m)
        off = off + plsc.all_reduce_population_count(m)
      return off

    zero = jnp.zeros((L,), jnp.int32)
    off = lax.fori_loop(0, ncnk // 2, pass_b, zero)
    cnt = jnp.max(off)

    sv1, _ = plsc.sort_key_val(vmax, iota, descending=True)
    sv2, _ = plsc.sort_key_val(vmin, iota, descending=True)
    sent = jnp.min(jnp.maximum(sv1, lax.rev(sv2, (0,))))
    imax_vec = iota | IMAX
    plsc.store_scatter(cv_t, [off + iota],
                       jnp.broadcast_to(sent, (L,)))
    plsc.store_scatter(ci_t, [off + iota], imax_vec)

    def pass_c(r, carry):
      tv, ti, bv, bi = carry
      sv, si = plsc.sort_key_val(cv_t[pl.ds(r * L, L)], ci_t[pl.ds(r * L, L)],
                                 descending=True)
      tv, ti = _merge16(tv, ti, sv, si, descending=True)
      bv, bi = _merge16(bv, bi, lax.rev(sv, (0,)), lax.rev(si, (0,)),
                        descending=False)
      return tv, ti, bv, bi

    i0 = jnp.full((L,), IMAX, jnp.int32)
    tv, ti, bv, bi = lax.fori_loop(
        0, (cnt + L - 1) // L, pass_c,
        (jnp.full((L,), -BIG, jnp.float32), i0,
         jnp.full((L,), BIG, jnp.float32), i0))

    pack_v[pl.ds(0, L)] = tv
    pack_v[pl.ds(L, L)] = plsc.bitcast(ti, jnp.float32)
    pack_v[pl.ds(2 * L, L)] = bv
    pack_v[pl.ds(3 * L, L)] = plsc.bitcast(bi, jnp.float32)
    pltpu.sync_copy(pack_v, stage_sh.at[pl.ds(sid * 4 * L, 4 * L)])
    plsc.subcore_barrier()

    @pl.when(sid == 0)
    def _():
      pltpu.sync_copy(stage_sh, all_v)

      def merge_w(w, carry):
        tv, ti, bv, bi = carry
        b0 = w * 4 * L
        xv_t = all_v[pl.ds(b0, L)]
        xi_t = plsc.bitcast(all_v[pl.ds(b0 + L, L)], jnp.int32)
        xv_b = all_v[pl.ds(b0 + 2 * L, L)]
        xi_b = plsc.bitcast(all_v[pl.ds(b0 + 3 * L, L)], jnp.int32)
        tv, ti = _merge16(tv, ti, xv_t, xi_t, descending=True)
        bv, bi = _merge16(bv, bi, xv_b, xi_b, descending=False)
        return tv, ti, bv, bi

      gtv, gti, gbv, gbi = lax.fori_loop(
          1, NS, merge_w,
          (all_v[pl.ds(0, L)], plsc.bitcast(all_v[pl.ds(L, L)], jnp.int32),
           all_v[pl.ds(2 * L, L)],
           plsc.bitcast(all_v[pl.ds(3 * L, L)], jnp.int32)))

      aout_v[pl.ds(0, L)] = gtv
      aout_v[pl.ds(L, L)] = lax.rev(gbv, (0,))
      idx_v[pl.ds(0, L)] = gti
      idx_v[pl.ds(L, L)] = lax.rev(gbi, (0,))

      pltpu.async_copy(feat_hbm.at[idx_v], rows_v, sem).wait()
      pltpu.sync_copy(aout_v, att_out_hbm)
      pltpu.sync_copy(rows_v, feat_out_hbm)


def kernel(x_features, x_attention):
  att = jnp.squeeze(x_attention, -1)
  sel_att, sel_feat = _minmax_select(att, x_features)
  return sel_att.reshape(2 * TOPK, 1), sel_feat


# speedup vs baseline: 1.1324x; 1.1324x over previous
import functools

import jax
import jax.numpy as jnp
from jax import lax
from jax.experimental import pallas as pl
from jax.experimental.pallas import tpu as pltpu
from jax.experimental.pallas import tpu_sc as plsc

N = 100000
D = 128
TOPK = 16
L = 16
NC = 2
NS = 16
PW = 6272
PWL = N - (NS - 1) * PW
CPW = PW // L
BIG = float(jnp.finfo(jnp.float32).max)
IMAX = 2**31 - 1

_mesh = plsc.VectorSubcoreMesh(core_axis_name="c", subcore_axis_name="s",
                               num_cores=1)


def _merge16(av, ai, xv, xi, descending):
  rv = lax.rev(xv, (0,))
  ri = lax.rev(xi, (0,))
  if descending:
    take = (av > rv) | ((av == rv) & (ai < ri))
  else:
    take = (av < rv) | ((av == rv) & (ai < ri))
  mv = jnp.where(take, av, rv)
  mi = jnp.where(take, ai, ri)
  sv, si = plsc.sort_key_val(mv, mi, descending=descending)
  return sv, si


@functools.partial(
    pl.kernel,
    out_type=(
        jax.ShapeDtypeStruct((2 * TOPK,), jnp.float32),
        jax.ShapeDtypeStruct((2 * TOPK, D), jnp.float32),
    ),
    mesh=_mesh,
    compiler_params=pltpu.CompilerParams(needs_layout_passes=False),
    scratch_types=[
        pltpu.VMEM((PW,), jnp.float32),
        pltpu.VMEM((PW + L,), jnp.float32),
        pltpu.VMEM((PW + L,), jnp.int32),
        pltpu.VMEM((PW + L,), jnp.float32),
        pltpu.VMEM((PW + L,), jnp.int32),
        pltpu.VMEM((4 * L,), jnp.float32),
        pltpu.VMEM((NS * 4 * L,), jnp.float32),
        pltpu.VMEM_SHARED((NS * 4 * L,), jnp.float32),
        pltpu.VMEM((2 * TOPK,), jnp.int32),
        pltpu.VMEM((2 * TOPK,), jnp.float32),
        pltpu.VMEM((2 * TOPK, D), jnp.float32),
        pltpu.SemaphoreType.DMA,
    ],
)
def _minmax_select(att_hbm, feat_hbm, att_out_hbm, feat_out_hbm,
                   att_v, cv_t, ci_t, cv_b, ci_b, pack_v, all_v, stage_sh,
                   idx_v, aout_v, rows_v, sem):
  cid = lax.axis_index("c")
  sid = lax.axis_index("s")

  @pl.when(cid == 0)
  def _():
    base = sid * PW
    ncnk = jnp.where(sid == NS - 1, PWL // L, CPW)
    pltpu.sync_copy(att_hbm.at[pl.ds(base, PWL)], att_v.at[pl.ds(0, PWL)])

    @pl.when(sid != NS - 1)
    def _copy_tail():
      pltpu.sync_copy(att_hbm.at[pl.ds(base + PWL, PW - PWL)],
                      att_v.at[pl.ds(PWL, PW - PWL)])

    iota = lax.iota(jnp.int32, L)
    fifteen = jnp.full((L,), 15, jnp.int32)

    @plsc.parallel_loop(
        0, ncnk // 2, unroll=8,
        carry=(jnp.full((L,), -BIG, jnp.float32),
               jnp.full((L,), BIG, jnp.float32)))
    def pass_a(c, carry):
      vmax, vmin = carry
      b0 = c * (2 * L)
      for u in range(2):
        v = att_v[pl.ds(b0 + u * L, L)]
        vmax = jnp.maximum(vmax, v)
        vmin = jnp.minimum(vmin, v)
      return vmax, vmin

    vmax, vmin = pass_a
    thr_t = jnp.min(vmax)
    thr_b = jnp.max(vmin)

    zero = jnp.zeros((L,), jnp.int32)

    @plsc.parallel_loop(0, ncnk // 2, unroll=4, carry=zero)
    def pass_b(c, off):
      b0 = c * (2 * L)
      for u in range(2):
        v = att_v[pl.ds(b0 + u * L, L)]
        idx = iota + (base + b0 + u * L)
        m = (v >= thr_t) | (v <= thr_b)
        pos = plsc.cumsum(m.astype(jnp.int32))
        iw = off + pos - 1
        plsc.store_scatter(cv_t, [iw], v, mask=m)
        plsc.store_scatter(ci_t, [iw], idx, mask=m)
        off = off + plsc.all_reduce_population_count(m)
      return off

    off = pass_b
    cnt = jnp.max(off)

    sv1, _ = plsc.sort_key_val(vmax, iota, descending=True)
    sv2, _ = plsc.sort_key_val(vmin, iota, descending=True)
    sent = jnp.min(jnp.maximum(sv1, lax.rev(sv2, (0,))))
    imax_vec = iota | IMAX
    plsc.store_scatter(cv_t, [off + iota],
                       jnp.broadcast_to(sent, (L,)))
    plsc.store_scatter(ci_t, [off + iota], imax_vec)

    def pass_c(r, carry):
      tv, ti, bv, bi = carry
      sv, si = plsc.sort_key_val(cv_t[pl.ds(r * L, L)], ci_t[pl.ds(r * L, L)],
                                 descending=True)
      tv, ti = _merge16(tv, ti, sv, si, descending=True)
      bv, bi = _merge16(bv, bi, lax.rev(sv, (0,)), lax.rev(si, (0,)),
                        descending=False)
      return tv, ti, bv, bi

    i0 = jnp.full((L,), IMAX, jnp.int32)
    tv, ti, bv, bi = lax.fori_loop(
        0, (cnt + L - 1) // L, pass_c,
        (jnp.full((L,), -BIG, jnp.float32), i0,
         jnp.full((L,), BIG, jnp.float32), i0))

    pack_v[pl.ds(0, L)] = tv
    pack_v[pl.ds(L, L)] = plsc.bitcast(ti, jnp.float32)
    pack_v[pl.ds(2 * L, L)] = bv
    pack_v[pl.ds(3 * L, L)] = plsc.bitcast(bi, jnp.float32)
    pltpu.sync_copy(pack_v, stage_sh.at[pl.ds(sid * 4 * L, 4 * L)])
    plsc.subcore_barrier()

    @pl.when(sid == 0)
    def _():
      pltpu.sync_copy(stage_sh, all_v)

      def merge_w(w, carry):
        tv, ti, bv, bi = carry
        b0 = w * 4 * L
        xv_t = all_v[pl.ds(b0, L)]
        xi_t = plsc.bitcast(all_v[pl.ds(b0 + L, L)], jnp.int32)
        xv_b = all_v[pl.ds(b0 + 2 * L, L)]
        xi_b = plsc.bitcast(all_v[pl.ds(b0 + 3 * L, L)], jnp.int32)
        tv, ti = _merge16(tv, ti, xv_t, xi_t, descending=True)
        bv, bi = _merge16(bv, bi, xv_b, xi_b, descending=False)
        return tv, ti, bv, bi

      gtv, gti, gbv, gbi = lax.fori_loop(
          1, NS, merge_w,
          (all_v[pl.ds(0, L)], plsc.bitcast(all_v[pl.ds(L, L)], jnp.int32),
           all_v[pl.ds(2 * L, L)],
           plsc.bitcast(all_v[pl.ds(3 * L, L)], jnp.int32)))

      aout_v[pl.ds(0, L)] = gtv
      aout_v[pl.ds(L, L)] = lax.rev(gbv, (0,))
      idx_v[pl.ds(0, L)] = gti
      idx_v[pl.ds(L, L)] = lax.rev(gbi, (0,))

      pltpu.async_copy(feat_hbm.at[idx_v], rows_v, sem).wait()
      pltpu.sync_copy(aout_v, att_out_hbm)
      pltpu.sync_copy(rows_v, feat_out_hbm)


def kernel(x_features, x_attention):
  att = jnp.squeeze(x_attention, -1)
  sel_att, sel_feat = _minmax_select(att, x_features)
  return sel_att.reshape(2 * TOPK, 1), sel_feat
